# baseline (speedup 1.0000x reference)
import jax
import jax.numpy as jnp
from jax import lax
from jax.experimental import pallas as pl
from jax.experimental.pallas import tpu as pltpu
from jax.experimental.pallas import tpu_sc as plsc

N_FEAT = 128
FH = 64
LANES = 16
N_CORES = 2
N_SUBCORES = 16
N_TILES = N_CORES * N_SUBCORES
CHUNK = 112
IBUF = 6
ROW_STRIDE = 624
ROWS_PER_TILE = 640


def _matmul_body(x_ref, wl_ref, wr_ref, b_ref, hp_ref, dense_ref):
    x = x_ref[...]
    h = jnp.dot(x, wl_ref[...], preferred_element_type=jnp.float32)
    hb = h.astype(jnp.bfloat16)
    lo = lax.bitcast_convert_type(hb[:, :FH], jnp.uint16).astype(jnp.uint32)
    hi = lax.bitcast_convert_type(hb[:, FH:], jnp.uint16).astype(jnp.uint32)
    hp_ref[...] = lax.bitcast_convert_type(lo | (hi << 16), jnp.int32)
    dense_ref[...] = (
        jnp.dot(x, wr_ref[...], preferred_element_type=jnp.float32) + b_ref[...]
    )


def _combine_body(p_ref, d_ref, o_ref):
    o_ref[...] = p_ref[0] + p_ref[1] + d_ref[...]


def _sc_body(cpt, hp_hbm, ipack_hbm, outp_hbm, acc, gbufs, sbufs, ips,
             gsem, ssem, isem):
    cid = lax.axis_index("c")
    sid = lax.axis_index("s")
    wid = cid * N_SUBCORES + sid
    chunk_base = wid * cpt

    def fetch_ipack(j, s):
        pltpu.async_copy(ipack_hbm.at[chunk_base + j], ips[s], isem[s])

    def wait_ipack(j, s):
        pltpu.make_async_copy(ipack_hbm.at[chunk_base + j], ips[s], isem[s]).wait()

    def start_gather(j, s, b):
        pltpu.async_copy(hp_hbm.at[ips[s].at[0]], gbufs[b], gsem[b])

    def wait_gather(j, s, b):
        pltpu.make_async_copy(hp_hbm.at[ips[s].at[0]], gbufs[b], gsem[b]).wait()

    def start_scatter(j, s, b):
        pltpu.async_copy(sbufs[b], acc.at[ips[s].at[1]], ssem[b], add=True)

    def wait_scatter(j, s, b):
        pltpu.make_async_copy(sbufs[b], acc.at[ips[s].at[1]], ssem[b]).wait()

    for j in range(4):
        fetch_ipack(j, j)

    def zero_row(e, _):
        for f in range(N_FEAT // LANES):
            sbufs[0][e, pl.ds(f * LANES, LANES)] = jnp.zeros((LANES,), jnp.float32)
        return 0
    lax.fori_loop(0, CHUNK, zero_row, 0)
    row_base = sid * ROW_STRIDE
    for k in range(ROWS_PER_TILE // CHUNK):
        pltpu.sync_copy(sbufs[0], acc.at[pl.ds(row_base + k * CHUNK, CHUNK)])
    rem = ROWS_PER_TILE % CHUNK
    if rem:
        nfull = ROWS_PER_TILE // CHUNK
        pltpu.sync_copy(sbufs[0].at[pl.ds(0, rem)],
                        acc.at[pl.ds(row_base + nfull * CHUNK, rem)])

    for j in range(2):
        wait_ipack(j, j)
        start_gather(j, j, j)
    plsc.subcore_barrier()

    n_groups = cpt // IBUF
    mask_hi = jnp.full((LANES,), -65536, jnp.int32)

    def step(g, _):
        for p in range(IBUF):
            i = g * IBUF + p
            b = p % 2
            s = p

            sj = (p + 4) % IBUF
            if p < 2:
                @pl.when(g > 0)
                def _():
                    wait_scatter(i - 2, sj, b)
                fetch_ipack(i + 4, sj)
            else:
                wait_scatter(i - 2, sj, b)

                @pl.when(g < n_groups - 1)
                def _():
                    fetch_ipack(i + 4, sj)

            wait_gather(i, s, b)

            def scale_group(q, _):
                ew16 = lax.bitcast_convert_type(
                    ips[s][2, pl.ds(q * LANES, LANES)], jnp.float32)
                for l in range(LANES):
                    w = jnp.full((LANES,), ew16[l], jnp.float32)
                    e = q * LANES + l
                    for f in range(FH // LANES):
                        v = gbufs[b][e, pl.ds(f * LANES, LANES)]
                        f_lo = lax.bitcast_convert_type(v << 16, jnp.float32)
                        f_hi = lax.bitcast_convert_type(v & mask_hi, jnp.float32)
                        sbufs[b][e, pl.ds(f * LANES, LANES)] = f_lo * w
                        sbufs[b][e, pl.ds(FH + f * LANES, LANES)] = f_hi * w
                return 0
            lax.fori_loop(0, CHUNK // LANES, scale_group, 0)
            start_scatter(i, s, b)

            s2 = (p + 2) % IBUF
            if p < 4:
                wait_ipack(i + 2, s2)
                start_gather(i + 2, s2, b)
            else:
                @pl.when(g < n_groups - 1)
                def _():
                    wait_ipack(i + 2, s2)
                    start_gather(i + 2, s2, b)
        return 0
    lax.fori_loop(0, n_groups, step, 0)

    for b in range(2):
        wait_scatter(cpt - 2 + b, (b + 4) % IBUF, b)
    plsc.subcore_barrier()

    pltpu.sync_copy(acc.at[pl.ds(row_base, ROWS_PER_TILE)],
                    outp_hbm.at[cid, pl.ds(row_base, ROWS_PER_TILE)])


def kernel(x, edge_index, edge_weight, W_l, W_r, bias):
    n, f = x.shape
    e = edge_weight.shape[0]
    src = edge_index[0].astype(jnp.int32)
    dst = edge_index[1].astype(jnp.int32)
    ew = edge_weight.astype(jnp.float32)

    cpt = -(-e // (N_TILES * CHUNK))
    cpt = -(-cpt // IBUF) * IBUF
    e_pad = N_TILES * cpt * CHUNK
    pad = e_pad - e
    if pad:
        src = jnp.pad(src, (0, pad))
        dst = jnp.pad(dst, (0, pad))
        ew = jnp.pad(ew, (0, pad))
    ipack = jnp.stack(
        [src.reshape(-1, CHUNK), dst.reshape(-1, CHUNK),
         lax.bitcast_convert_type(ew, jnp.int32).reshape(-1, CHUNK)], axis=1)

    blk = 2000
    grid = n // blk
    hp, dense = pl.pallas_call(
        _matmul_body,
        grid=(grid,),
        in_specs=[
            pl.BlockSpec((blk, f), lambda i: (i, 0)),
            pl.BlockSpec((f, N_FEAT), lambda i: (0, 0)),
            pl.BlockSpec((f, N_FEAT), lambda i: (0, 0)),
            pl.BlockSpec((1, N_FEAT), lambda i: (0, 0)),
        ],
        out_specs=[
            pl.BlockSpec((blk, FH), lambda i: (i, 0)),
            pl.BlockSpec((blk, N_FEAT), lambda i: (i, 0)),
        ],
        out_shape=[
            jax.ShapeDtypeStruct((n, FH), jnp.int32),
            jax.ShapeDtypeStruct((n, N_FEAT), jnp.float32),
        ],
    )(x, W_l, W_r, bias.reshape(1, N_FEAT))

    mesh = plsc.VectorSubcoreMesh(core_axis_name="c", subcore_axis_name="s")

    def sc_entry(hp_a, ipack_a, outp_a, acc, gb0, gb1, sb0, sb1,
                 i0, i1, i2, i3, i4, i5,
                 g0, g1, s0, s1, q0, q1, q2, q3, q4, q5):
        _sc_body(cpt, hp_a, ipack_a, outp_a, acc,
                 (gb0, gb1), (sb0, sb1), (i0, i1, i2, i3, i4, i5),
                 (g0, g1), (s0, s1), (q0, q1, q2, q3, q4, q5))

    sc_fn = pl.kernel(
        sc_entry,
        out_type=jax.ShapeDtypeStruct((N_CORES, n, N_FEAT), jnp.float32),
        mesh=mesh,
        scratch_types=(
            [pltpu.VMEM_SHARED((n, N_FEAT), jnp.float32)]
            + [pltpu.VMEM((CHUNK, FH), jnp.int32)] * 2
            + [pltpu.VMEM((CHUNK, N_FEAT), jnp.float32)] * 2
            + [pltpu.VMEM((3, CHUNK), jnp.int32)] * IBUF
            + [pltpu.SemaphoreType.DMA] * (4 + IBUF)
        ),
        compiler_params=pltpu.CompilerParams(use_tc_tiling_on_sc=False),
    )
    outp = sc_fn(hp, ipack)

    out = pl.pallas_call(
        _combine_body,
        grid=(grid,),
        in_specs=[
            pl.BlockSpec((N_CORES, blk, N_FEAT), lambda i: (0, i, 0)),
            pl.BlockSpec((blk, N_FEAT), lambda i: (i, 0)),
        ],
        out_specs=pl.BlockSpec((blk, N_FEAT), lambda i: (i, 0)),
        out_shape=jax.ShapeDtypeStruct((n, N_FEAT), jnp.float32),
    )(outp, dense)
    return out


# speedup vs baseline: 1.2155x; 1.2155x over previous
import jax
import jax.numpy as jnp
from jax import lax
from jax.experimental import pallas as pl
from jax.experimental.pallas import tpu as pltpu
from jax.experimental.pallas import tpu_sc as plsc

N_FEAT = 128
LANES = 16
N_CORES = 2
N_SUBCORES = 16
N_TILES = N_CORES * N_SUBCORES
CHUNK = 112
NBUF = 3
IBUF = 6
ROW_STRIDE = 624
ROWS_PER_TILE = 640


def _matmul_body(x_ref, wl_ref, wr_ref, b_ref, h_ref, dense_ref):
    x = x_ref[...]
    h_ref[...] = jnp.dot(x, wl_ref[...], preferred_element_type=jnp.float32)
    dense_ref[...] = (
        jnp.dot(x, wr_ref[...], preferred_element_type=jnp.float32) + b_ref[...]
    )


def _combine_body(p_ref, d_ref, o_ref):
    o_ref[...] = p_ref[0] + p_ref[1] + d_ref[...]


def _sc_body(cpt0, cpt1, h_hbm, ipack_hbm, outp_hbm, acc, rows, ips,
             gsem, ssem, isem):
    cid = lax.axis_index("c")
    sid = lax.axis_index("s")
    cpt = jnp.where(cid == 0, cpt0, cpt1)
    n_groups_dyn = jnp.where(cid == 0, cpt0 // IBUF, cpt1 // IBUF)
    chunk_base = jnp.where(cid == 0, sid * cpt0,
                           N_SUBCORES * cpt0 + sid * cpt1)

    def fetch_ipack(j, s):
        pltpu.async_copy(ipack_hbm.at[chunk_base + j], ips[s], isem[s])

    def wait_ipack(j, s):
        pltpu.make_async_copy(ipack_hbm.at[chunk_base + j], ips[s], isem[s]).wait()

    def start_gather(j, s, b):
        pltpu.async_copy(h_hbm.at[ips[s].at[0]], rows[b], gsem[b])

    def wait_gather(j, s, b):
        pltpu.make_async_copy(h_hbm.at[ips[s].at[0]], rows[b], gsem[b]).wait()

    def start_scatter(j, s, b):
        pltpu.async_copy(rows[b], acc.at[ips[s].at[1]], ssem[b], add=True)

    def wait_scatter(j, s, b):
        pltpu.make_async_copy(rows[b], acc.at[ips[s].at[1]], ssem[b]).wait()

    for j in range(4):
        fetch_ipack(j, j)

    def zero_row(e, _):
        for f in range(N_FEAT // LANES):
            rows[0][e, pl.ds(f * LANES, LANES)] = jnp.zeros((LANES,), jnp.float32)
        return 0
    lax.fori_loop(0, CHUNK, zero_row, 0)
    row_base = sid * ROW_STRIDE
    for k in range(ROWS_PER_TILE // CHUNK):
        pltpu.sync_copy(rows[0], acc.at[pl.ds(row_base + k * CHUNK, CHUNK)])
    rem = ROWS_PER_TILE % CHUNK
    if rem:
        nfull = ROWS_PER_TILE // CHUNK
        pltpu.sync_copy(rows[0].at[pl.ds(0, rem)],
                        acc.at[pl.ds(row_base + nfull * CHUNK, rem)])

    for j in range(2):
        wait_ipack(j, j)
        start_gather(j, j, j)
    plsc.subcore_barrier()

    n_groups = n_groups_dyn

    def step(g, _):
        for p in range(IBUF):
            i = g * IBUF + p
            b = p % NBUF
            s = p

            sj = (p + 4) % IBUF
            if p < 2:
                fetch_ipack(i + 4, sj)
            else:
                @pl.when(g < n_groups - 1)
                def _():
                    fetch_ipack(i + 4, sj)

            wait_gather(i, s, b)

            def scale_group(q, _):
                ew16 = lax.bitcast_convert_type(
                    ips[s][2, pl.ds(q * LANES, LANES)], jnp.float32)
                for l in range(LANES):
                    w = jnp.full((LANES,), ew16[l], jnp.float32)
                    for f in range(N_FEAT // LANES):
                        sl = pl.ds(f * LANES, LANES)
                        rows[b][q * LANES + l, sl] = rows[b][q * LANES + l, sl] * w
                return 0
            lax.fori_loop(0, CHUNK // LANES, scale_group, 0)
            start_scatter(i, s, b)

            t = (p + 2) % NBUF
            s2 = (p + 2) % IBUF
            sp = (p + 5) % IBUF
            if p < 4:
                if p == 0:
                    @pl.when(g > 0)
                    def _():
                        wait_scatter(i - 1, sp, t)
                else:
                    wait_scatter(i - 1, sp, t)
                wait_ipack(i + 2, s2)
                start_gather(i + 2, s2, t)
            else:
                @pl.when(g < n_groups - 1)
                def _():
                    wait_scatter(i - 1, sp, t)
                    wait_ipack(i + 2, s2)
                    start_gather(i + 2, s2, t)
        return 0
    lax.fori_loop(0, n_groups, step, 0)

    for b in range(NBUF):
        wait_scatter(cpt - NBUF + b, (b + NBUF) % IBUF, b)
    plsc.subcore_barrier()

    pltpu.sync_copy(acc.at[pl.ds(row_base, ROWS_PER_TILE)],
                    outp_hbm.at[cid, pl.ds(row_base, ROWS_PER_TILE)])


def kernel(x, edge_index, edge_weight, W_l, W_r, bias):
    n, f = x.shape
    e = edge_weight.shape[0]
    src = edge_index[0].astype(jnp.int32)
    dst = edge_index[1].astype(jnp.int32)
    ew = edge_weight.astype(jnp.float32)

    unit = N_SUBCORES * CHUNK
    cpt1 = -(-e // (3 * unit * IBUF)) * IBUF
    cpt0 = 2 * cpt1
    e_pad = unit * (cpt0 + cpt1)
    pad = e_pad - e
    if pad:
        src = jnp.pad(src, (0, pad))
        dst = jnp.pad(dst, (0, pad))
        ew = jnp.pad(ew, (0, pad))
    ipack = jnp.stack(
        [src.reshape(-1, CHUNK), dst.reshape(-1, CHUNK),
         lax.bitcast_convert_type(ew, jnp.int32).reshape(-1, CHUNK)], axis=1)

    blk = 2000
    grid = n // blk
    h, dense = pl.pallas_call(
        _matmul_body,
        grid=(grid,),
        in_specs=[
            pl.BlockSpec((blk, f), lambda i: (i, 0)),
            pl.BlockSpec((f, N_FEAT), lambda i: (0, 0)),
            pl.BlockSpec((f, N_FEAT), lambda i: (0, 0)),
            pl.BlockSpec((1, N_FEAT), lambda i: (0, 0)),
        ],
        out_specs=[
            pl.BlockSpec((blk, N_FEAT), lambda i: (i, 0)),
            pl.BlockSpec((blk, N_FEAT), lambda i: (i, 0)),
        ],
        out_shape=[
            jax.ShapeDtypeStruct((n, N_FEAT), jnp.float32),
            jax.ShapeDtypeStruct((n, N_FEAT), jnp.float32),
        ],
    )(x, W_l, W_r, bias.reshape(1, N_FEAT))

    mesh = plsc.VectorSubcoreMesh(core_axis_name="c", subcore_axis_name="s")

    def sc_entry(h_a, ipack_a, outp_a, acc, r0, r1, r2, i0, i1, i2, i3, i4, i5,
                 g0, g1, g2, s0, s1, s2, q0, q1, q2, q3, q4, q5):
        _sc_body(cpt0, cpt1, h_a, ipack_a, outp_a, acc,
                 (r0, r1, r2), (i0, i1, i2, i3, i4, i5),
                 (g0, g1, g2), (s0, s1, s2), (q0, q1, q2, q3, q4, q5))

    sc_fn = pl.kernel(
        sc_entry,
        out_type=jax.ShapeDtypeStruct((N_CORES, n, N_FEAT), jnp.float32),
        mesh=mesh,
        scratch_types=(
            [pltpu.VMEM_SHARED((n, N_FEAT), jnp.float32)]
            + [pltpu.VMEM((CHUNK, N_FEAT), jnp.float32)] * NBUF
            + [pltpu.VMEM((3, CHUNK), jnp.int32)] * IBUF
            + [pltpu.SemaphoreType.DMA] * (2 * NBUF + IBUF)
        ),
    )
    outp = sc_fn(h, ipack)

    out = pl.pallas_call(
        _combine_body,
        grid=(grid,),
        in_specs=[
            pl.BlockSpec((N_CORES, blk, N_FEAT), lambda i: (0, i, 0)),
            pl.BlockSpec((blk, N_FEAT), lambda i: (i, 0)),
        ],
        out_specs=pl.BlockSpec((blk, N_FEAT), lambda i: (i, 0)),
        out_shape=jax.ShapeDtypeStruct((n, N_FEAT), jnp.float32),
    )(outp, dense)
    return out


# speedup vs baseline: 1.2679x; 1.0431x over previous
import jax
import jax.numpy as jnp
from jax import lax
from jax.experimental import pallas as pl
from jax.experimental.pallas import tpu as pltpu
from jax.experimental.pallas import tpu_sc as plsc

N_FEAT = 128
LANES = 16
N_CORES = 2
N_SUBCORES = 16
N_TILES = N_CORES * N_SUBCORES
CHUNK = 112
NBUF = 3
IBUF = 6
ROW_STRIDE = 624
ROWS_PER_TILE = 640


def _matmul_body(x_ref, wl_ref, wr_ref, b_ref, h_ref, dense_ref):
    x = x_ref[...]
    h_ref[...] = jnp.dot(x, wl_ref[...], preferred_element_type=jnp.float32)
    dense_ref[...] = (
        jnp.dot(x, wr_ref[...], preferred_element_type=jnp.float32) + b_ref[...]
    )


def _combine_body(p_ref, d_ref, o_ref):
    o_ref[...] = p_ref[0] + p_ref[1] + d_ref[...]


def _sc_body(cpt0, cpt1, h_hbm, ipack_hbm, outp_hbm, acc, rows, ips,
             gsem, ssem, isem):
    cid = lax.axis_index("c")
    sid = lax.axis_index("s")
    cpt = jnp.where(cid == 0, cpt0, cpt1)
    n_groups_dyn = jnp.where(cid == 0, cpt0 // IBUF, cpt1 // IBUF)
    chunk_base = jnp.where(cid == 0, sid * cpt0,
                           N_SUBCORES * cpt0 + sid * cpt1)

    def fetch_ipack(j, s):
        pltpu.async_copy(ipack_hbm.at[chunk_base + j], ips[s], isem[s])

    def wait_ipack(j, s):
        pltpu.make_async_copy(ipack_hbm.at[chunk_base + j], ips[s], isem[s]).wait()

    def start_gather(j, s, b):
        pltpu.async_copy(h_hbm.at[ips[s].at[0]], rows[b], gsem[b])

    def wait_gather(j, s, b):
        pltpu.make_async_copy(h_hbm.at[ips[s].at[0]], rows[b], gsem[b]).wait()

    def start_scatter(j, s, b):
        pltpu.async_copy(rows[b], acc.at[ips[s].at[1]], ssem[b], add=True)

    def wait_scatter(j, s, b):
        pltpu.make_async_copy(rows[b], acc.at[ips[s].at[1]], ssem[b]).wait()

    for j in range(4):
        fetch_ipack(j, j)

    def zero_row(e, _):
        for f in range(N_FEAT // LANES):
            rows[0][e, pl.ds(f * LANES, LANES)] = jnp.zeros((LANES,), jnp.float32)
        return 0
    lax.fori_loop(0, CHUNK, zero_row, 0)
    row_base = sid * ROW_STRIDE
    for k in range(ROWS_PER_TILE // CHUNK):
        pltpu.sync_copy(rows[0], acc.at[pl.ds(row_base + k * CHUNK, CHUNK)])
    rem = ROWS_PER_TILE % CHUNK
    if rem:
        nfull = ROWS_PER_TILE // CHUNK
        pltpu.sync_copy(rows[0].at[pl.ds(0, rem)],
                        acc.at[pl.ds(row_base + nfull * CHUNK, rem)])

    for j in range(2):
        wait_ipack(j, j)
        start_gather(j, j, j)
    plsc.subcore_barrier()

    n_groups = n_groups_dyn

    def step(g, _):
        for p in range(IBUF):
            i = g * IBUF + p
            b = p % NBUF
            s = p

            sj = (p + 4) % IBUF
            if p < 2:
                fetch_ipack(i + 4, sj)
            else:
                @pl.when(g < n_groups - 1)
                def _():
                    fetch_ipack(i + 4, sj)

            wait_gather(i, s, b)

            def scale_group(q, _):
                ew16 = lax.bitcast_convert_type(
                    ips[s][2, pl.ds(q * LANES, LANES)], jnp.float32)
                for l in range(LANES):
                    w = jnp.full((LANES,), ew16[l], jnp.float32)
                    for f in range(N_FEAT // LANES):
                        sl = pl.ds(f * LANES, LANES)
                        rows[b][q * LANES + l, sl] = rows[b][q * LANES + l, sl] * w
                return 0
            lax.fori_loop(0, CHUNK // LANES, scale_group, 0)
            start_scatter(i, s, b)

            t = (p + 2) % NBUF
            s2 = (p + 2) % IBUF
            sp = (p + 5) % IBUF
            if p < 4:
                if p == 0:
                    @pl.when(g > 0)
                    def _():
                        wait_scatter(i - 1, sp, t)
                else:
                    wait_scatter(i - 1, sp, t)
                wait_ipack(i + 2, s2)
                start_gather(i + 2, s2, t)
            else:
                @pl.when(g < n_groups - 1)
                def _():
                    wait_scatter(i - 1, sp, t)
                    wait_ipack(i + 2, s2)
                    start_gather(i + 2, s2, t)
        return 0
    lax.fori_loop(0, n_groups, step, 0)

    for b in range(NBUF):
        wait_scatter(cpt - NBUF + b, (b + NBUF) % IBUF, b)
    plsc.subcore_barrier()

    pltpu.sync_copy(acc.at[pl.ds(row_base, ROWS_PER_TILE)],
                    outp_hbm.at[cid, pl.ds(row_base, ROWS_PER_TILE)])


def kernel(x, edge_index, edge_weight, W_l, W_r, bias):
    n, f = x.shape
    e = edge_weight.shape[0]
    src = edge_index[0].astype(jnp.int32)
    dst = edge_index[1].astype(jnp.int32)
    ew = edge_weight.astype(jnp.float32)

    unit = N_SUBCORES * CHUNK
    cpt_tot = 3 * (-(-e // (3 * unit * IBUF)) * IBUF)
    cpt1 = max(IBUF, cpt_tot * 4 // (15 * IBUF) * IBUF)
    cpt0 = cpt_tot - cpt1
    e_pad = unit * cpt_tot
    pad = e_pad - e
    if pad:
        src = jnp.pad(src, (0, pad))
        dst = jnp.pad(dst, (0, pad))
        ew = jnp.pad(ew, (0, pad))
    ipack = jnp.stack(
        [src.reshape(-1, CHUNK), dst.reshape(-1, CHUNK),
         lax.bitcast_convert_type(ew, jnp.int32).reshape(-1, CHUNK)], axis=1)

    blk = 2000
    grid = n // blk
    h, dense = pl.pallas_call(
        _matmul_body,
        grid=(grid,),
        in_specs=[
            pl.BlockSpec((blk, f), lambda i: (i, 0)),
            pl.BlockSpec((f, N_FEAT), lambda i: (0, 0)),
            pl.BlockSpec((f, N_FEAT), lambda i: (0, 0)),
            pl.BlockSpec((1, N_FEAT), lambda i: (0, 0)),
        ],
        out_specs=[
            pl.BlockSpec((blk, N_FEAT), lambda i: (i, 0)),
            pl.BlockSpec((blk, N_FEAT), lambda i: (i, 0)),
        ],
        out_shape=[
            jax.ShapeDtypeStruct((n, N_FEAT), jnp.float32),
            jax.ShapeDtypeStruct((n, N_FEAT), jnp.float32),
        ],
    )(x, W_l, W_r, bias.reshape(1, N_FEAT))

    mesh = plsc.VectorSubcoreMesh(core_axis_name="c", subcore_axis_name="s")

    def sc_entry(h_a, ipack_a, outp_a, acc, r0, r1, r2, i0, i1, i2, i3, i4, i5,
                 g0, g1, g2, s0, s1, s2, q0, q1, q2, q3, q4, q5):
        _sc_body(cpt0, cpt1, h_a, ipack_a, outp_a, acc,
                 (r0, r1, r2), (i0, i1, i2, i3, i4, i5),
                 (g0, g1, g2), (s0, s1, s2), (q0, q1, q2, q3, q4, q5))

    sc_fn = pl.kernel(
        sc_entry,
        out_type=jax.ShapeDtypeStruct((N_CORES, n, N_FEAT), jnp.float32),
        mesh=mesh,
        scratch_types=(
            [pltpu.VMEM_SHARED((n, N_FEAT), jnp.float32)]
            + [pltpu.VMEM((CHUNK, N_FEAT), jnp.float32)] * NBUF
            + [pltpu.VMEM((3, CHUNK), jnp.int32)] * IBUF
            + [pltpu.SemaphoreType.DMA] * (2 * NBUF + IBUF)
        ),
    )
    outp = sc_fn(h, ipack)

    out = pl.pallas_call(
        _combine_body,
        grid=(grid,),
        in_specs=[
            pl.BlockSpec((N_CORES, blk, N_FEAT), lambda i: (0, i, 0)),
            pl.BlockSpec((blk, N_FEAT), lambda i: (i, 0)),
        ],
        out_specs=pl.BlockSpec((blk, N_FEAT), lambda i: (i, 0)),
        out_shape=jax.ShapeDtypeStruct((n, N_FEAT), jnp.float32),
    )(outp, dense)
    return out


# speedup vs baseline: 1.3137x; 1.0361x over previous
import jax
import jax.numpy as jnp
from jax import lax
from jax.experimental import pallas as pl
from jax.experimental.pallas import tpu as pltpu
from jax.experimental.pallas import tpu_sc as plsc

N_FEAT = 128
LANES = 16
N_CORES = 2
N_SUBCORES = 16
N_TILES = N_CORES * N_SUBCORES
CHUNK = 112
NBUF = 3
IBUF = 6
ROW_STRIDE = 624
ROWS_PER_TILE = 640


def _matmul_body(x_ref, wl_ref, wr_ref, b_ref, h_ref, dense_ref):
    x = x_ref[...]
    h_ref[...] = jnp.dot(x, wl_ref[...], preferred_element_type=jnp.float32)
    dense_ref[...] = (
        jnp.dot(x, wr_ref[...], preferred_element_type=jnp.float32) + b_ref[...]
    )


def _combine_body(p_ref, d_ref, o_ref):
    o_ref[...] = p_ref[0] + p_ref[1] + d_ref[...]


def _sc_body(cpt0, cpt1, h_hbm, ipack_hbm, outp_hbm, acc, rows, ips,
             gsem, ssem, isem):
    cid = lax.axis_index("c")
    sid = lax.axis_index("s")
    cpt = jnp.where(cid == 0, cpt0, cpt1)
    n_groups_dyn = jnp.where(cid == 0, cpt0 // IBUF, cpt1 // IBUF)
    chunk_base = jnp.where(cid == 0, sid * cpt0,
                           N_SUBCORES * cpt0 + sid * cpt1)

    def fetch_ipack(j, s):
        pltpu.async_copy(ipack_hbm.at[chunk_base + j], ips[s], isem[s])

    def wait_ipack(j, s):
        pltpu.make_async_copy(ipack_hbm.at[chunk_base + j], ips[s], isem[s]).wait()

    def start_gather(j, s, b):
        pltpu.async_copy(h_hbm.at[ips[s].at[0]], rows[b], gsem[b])

    def wait_gather(j, s, b):
        pltpu.make_async_copy(h_hbm.at[ips[s].at[0]], rows[b], gsem[b]).wait()

    def start_scatter(j, s, b):
        pltpu.async_copy(rows[b], acc.at[ips[s].at[1]], ssem[b], add=True)

    def wait_scatter(j, s, b):
        pltpu.make_async_copy(rows[b], acc.at[ips[s].at[1]], ssem[b]).wait()

    for j in range(4):
        fetch_ipack(j, j)

    def zero_row(e, _):
        for f in range(N_FEAT // LANES):
            rows[0][e, pl.ds(f * LANES, LANES)] = jnp.zeros((LANES,), jnp.float32)
        return 0
    lax.fori_loop(0, CHUNK, zero_row, 0)
    row_base = sid * ROW_STRIDE
    for k in range(ROWS_PER_TILE // CHUNK):
        pltpu.sync_copy(rows[0], acc.at[pl.ds(row_base + k * CHUNK, CHUNK)])
    rem = ROWS_PER_TILE % CHUNK
    if rem:
        nfull = ROWS_PER_TILE // CHUNK
        pltpu.sync_copy(rows[0].at[pl.ds(0, rem)],
                        acc.at[pl.ds(row_base + nfull * CHUNK, rem)])

    for j in range(2):
        wait_ipack(j, j)
        start_gather(j, j, j)
    plsc.subcore_barrier()

    n_groups = n_groups_dyn

    def step(g, _):
        for p in range(IBUF):
            i = g * IBUF + p
            b = p % NBUF
            s = p

            sj = (p + 4) % IBUF
            if p < 2:
                fetch_ipack(i + 4, sj)
            else:
                @pl.when(g < n_groups - 1)
                def _():
                    fetch_ipack(i + 4, sj)

            wait_gather(i, s, b)

            def scale_group(q, _):
                ew16 = lax.bitcast_convert_type(
                    ips[s][2, pl.ds(q * LANES, LANES)], jnp.float32)
                for l in range(LANES):
                    w = jnp.full((LANES,), ew16[l], jnp.float32)
                    for f in range(N_FEAT // LANES):
                        sl = pl.ds(f * LANES, LANES)
                        rows[b][q * LANES + l, sl] = rows[b][q * LANES + l, sl] * w
                return 0
            lax.fori_loop(0, CHUNK // LANES, scale_group, 0)
            start_scatter(i, s, b)

            t = (p + 2) % NBUF
            s2 = (p + 2) % IBUF
            sp = (p + 5) % IBUF
            if p < 4:
                if p == 0:
                    @pl.when(g > 0)
                    def _():
                        wait_scatter(i - 1, sp, t)
                else:
                    wait_scatter(i - 1, sp, t)
                wait_ipack(i + 2, s2)
                start_gather(i + 2, s2, t)
            else:
                @pl.when(g < n_groups - 1)
                def _():
                    wait_scatter(i - 1, sp, t)
                    wait_ipack(i + 2, s2)
                    start_gather(i + 2, s2, t)
        return 0
    lax.fori_loop(0, n_groups, step, 0)

    for b in range(NBUF):
        wait_scatter(cpt - NBUF + b, (b + NBUF) % IBUF, b)
    plsc.subcore_barrier()

    pltpu.sync_copy(acc.at[pl.ds(row_base, ROWS_PER_TILE)],
                    outp_hbm.at[cid, pl.ds(row_base, ROWS_PER_TILE)])


def kernel(x, edge_index, edge_weight, W_l, W_r, bias):
    n, f = x.shape
    e = edge_weight.shape[0]
    src = edge_index[0].astype(jnp.int32)
    dst = edge_index[1].astype(jnp.int32)
    ew = edge_weight.astype(jnp.float32)

    unit = N_SUBCORES * CHUNK
    cpt_tot = 3 * (-(-e // (3 * unit * IBUF)) * IBUF)
    cpt1 = max(IBUF, cpt_tot * 3 // (15 * IBUF) * IBUF)
    cpt0 = cpt_tot - cpt1
    e_pad = unit * cpt_tot
    pad = e_pad - e
    if pad:
        src = jnp.pad(src, (0, pad))
        dst = jnp.pad(dst, (0, pad))
        ew = jnp.pad(ew, (0, pad))
    ipack = jnp.stack(
        [src.reshape(-1, CHUNK), dst.reshape(-1, CHUNK),
         lax.bitcast_convert_type(ew, jnp.int32).reshape(-1, CHUNK)], axis=1)

    blk = 2000
    grid = n // blk
    h, dense = pl.pallas_call(
        _matmul_body,
        grid=(grid,),
        in_specs=[
            pl.BlockSpec((blk, f), lambda i: (i, 0)),
            pl.BlockSpec((f, N_FEAT), lambda i: (0, 0)),
            pl.BlockSpec((f, N_FEAT), lambda i: (0, 0)),
            pl.BlockSpec((1, N_FEAT), lambda i: (0, 0)),
        ],
        out_specs=[
            pl.BlockSpec((blk, N_FEAT), lambda i: (i, 0)),
            pl.BlockSpec((blk, N_FEAT), lambda i: (i, 0)),
        ],
        out_shape=[
            jax.ShapeDtypeStruct((n, N_FEAT), jnp.float32),
            jax.ShapeDtypeStruct((n, N_FEAT), jnp.float32),
        ],
    )(x, W_l, W_r, bias.reshape(1, N_FEAT))

    mesh = plsc.VectorSubcoreMesh(core_axis_name="c", subcore_axis_name="s")

    def sc_entry(h_a, ipack_a, outp_a, acc, r0, r1, r2, i0, i1, i2, i3, i4, i5,
                 g0, g1, g2, s0, s1, s2, q0, q1, q2, q3, q4, q5):
        _sc_body(cpt0, cpt1, h_a, ipack_a, outp_a, acc,
                 (r0, r1, r2), (i0, i1, i2, i3, i4, i5),
                 (g0, g1, g2), (s0, s1, s2), (q0, q1, q2, q3, q4, q5))

    sc_fn = pl.kernel(
        sc_entry,
        out_type=jax.ShapeDtypeStruct((N_CORES, n, N_FEAT), jnp.float32),
        mesh=mesh,
        scratch_types=(
            [pltpu.VMEM_SHARED((n, N_FEAT), jnp.float32)]
            + [pltpu.VMEM((CHUNK, N_FEAT), jnp.float32)] * NBUF
            + [pltpu.VMEM((3, CHUNK), jnp.int32)] * IBUF
            + [pltpu.SemaphoreType.DMA] * (2 * NBUF + IBUF)
        ),
    )
    outp = sc_fn(h, ipack)

    out = pl.pallas_call(
        _combine_body,
        grid=(grid,),
        in_specs=[
            pl.BlockSpec((N_CORES, blk, N_FEAT), lambda i: (0, i, 0)),
            pl.BlockSpec((blk, N_FEAT), lambda i: (i, 0)),
        ],
        out_specs=pl.BlockSpec((blk, N_FEAT), lambda i: (i, 0)),
        out_shape=jax.ShapeDtypeStruct((n, N_FEAT), jnp.float32),
    )(outp, dense)
    return out
